# baseline (speedup 1.0000x reference)
import functools

import jax
import jax.numpy as jnp
from jax import lax
from jax.experimental import pallas as pl
from jax.experimental.pallas import tpu as pltpu
from jax.experimental.pallas import tpu_sc as plsc

B = 16384
D = 32
L = 16
NC = 2
NS = 16
NW = NC * NS
BPW = B // NW
NV = 1000000
W = 128
NWORDS = BPW * D
NDMA = NWORDS // W


@functools.partial(
    pl.kernel,
    out_type=jax.ShapeDtypeStruct((B,), jnp.float32),
    mesh=plsc.VectorSubcoreMesh(core_axis_name="c", subcore_axis_name="s"),
    compiler_params=pltpu.CompilerParams(
        needs_layout_passes=False, use_tc_tiling_on_sc=False),
    scratch_types=[
        pltpu.VMEM((BPW,), jnp.int32),
        pltpu.VMEM((BPW,), jnp.int32),
        pltpu.VMEM((NWORDS,), jnp.int32),
        pltpu.VMEM((NWORDS,), jnp.int32),
        pltpu.VMEM((NWORDS,), jnp.float32),
        pltpu.VMEM((NWORDS,), jnp.float32),
        pltpu.VMEM((BPW,), jnp.float32),
        pltpu.SemaphoreType.DMA,
        pltpu.SemaphoreType.DMA,
    ],
)
def _cf_sc(user_hbm, item_hbm, uflat_hbm, iflat_hbm, out_hbm,
           uidx, iidx, uwords, iwords, uvals, ivals, outv, sem_u, sem_i):
    wid = lax.axis_index("s") * NC + lax.axis_index("c")
    base = wid * BPW
    pltpu.sync_copy(user_hbm.at[pl.ds(base, BPW)], uidx)
    pltpu.sync_copy(item_hbm.at[pl.ds(base, BPW)], iidx)

    def build(c, carry):
        cbase = pl.multiple_of(c * L, L)
        u = uidx[pl.ds(cbase, L)]
        v = iidx[pl.ds(cbase, L)]
        for d in range(D):
            uwords[pl.ds(d * BPW + cbase, L)] = u + d * NV
            iwords[pl.ds(d * BPW + cbase, L)] = v + d * NV
        return carry

    lax.fori_loop(0, BPW // L, build, 0)

    def fire(k, carry):
        kbase = pl.multiple_of(k * W, W)
        pltpu.async_copy(uflat_hbm.at[uwords.at[pl.ds(kbase, W)]],
                         uvals.at[pl.ds(kbase, W)], sem_u)
        pltpu.async_copy(iflat_hbm.at[iwords.at[pl.ds(kbase, W)]],
                         ivals.at[pl.ds(kbase, W)], sem_i)
        return carry

    lax.fori_loop(0, NDMA, fire, 0)
    pltpu.make_async_copy(uflat_hbm.at[pl.ds(0, NWORDS)], uvals, sem_u).wait()
    pltpu.make_async_copy(iflat_hbm.at[pl.ds(0, NWORDS)], ivals, sem_i).wait()

    def group(g, carry):
        gbase = pl.multiple_of(g * L, L)
        acc = jnp.zeros((L,), jnp.float32)
        for d in range(D):
            u = uvals[pl.ds(d * BPW + gbase, L)]
            v = ivals[pl.ds(d * BPW + gbase, L)]
            acc = acc + u * v
        outv[pl.ds(gbase, L)] = acc
        return carry

    lax.fori_loop(0, BPW // L, group, 0)
    pltpu.sync_copy(outv, out_hbm.at[pl.ds(base, BPW)])


def kernel(user, item, user_emb, item_emb):
    uflat = user_emb.T.reshape(NV * D)
    iflat = item_emb.T.reshape(NV * D)
    return _cf_sc(user, item, uflat, iflat)
